# baseline (speedup 1.0000x reference)
import functools

import jax
import jax.numpy as jnp
from jax import lax
from jax.experimental import pallas as pl
from jax.experimental.pallas import tpu as pltpu
from jax.experimental.pallas import tpu_sc as plsc

_NC = 2
_NS = 16
_NW = _NC * _NS
_L = 16

_B = 16384
_PER_W = _B // _NW
_JROWS = _PER_W // 128
_HALF = _PER_W // 2
_HCH = _HALF // _L
_DIM = 3


def _sqrt16(z):
    zi = lax.bitcast_convert_type(z, jnp.int32)
    y = lax.bitcast_convert_type(jnp.int32(0x5F3759DF) - (zi >> 1), jnp.float32)
    for _ in range(3):
        y = y * (1.5 - 0.5 * z * y * y)
    return z * y


def _body(leg_hbm, vote_hbm, x_hbm, a_hbm, b_hbm, out_hbm,
          lv, vv, xi, aj, bj, ov, sem):
    wid = lax.axis_index("s") * _NC + lax.axis_index("c")
    base = wid * _PER_W

    pltpu.sync_copy(leg_hbm.at[pl.ds(base, _PER_W)], lv)
    for j in range(_JROWS):
        pltpu.sync_copy(vote_hbm.at[pl.ds(base + j * 128, 128)], vv.at[j])

    bcopies = [
        pltpu.async_copy(b_hbm.at[vv.at[j]], bj.at[pl.ds(j * 128, 128)], sem)
        for j in range(_JROWS)
    ]

    k0 = jnp.zeros((_L,), jnp.int32)
    k1 = jnp.full((_L,), 1, jnp.int32)
    k2 = jnp.full((_L,), 2, jnp.int32)
    for h in range(2):
        hb = h * _HALF
        copies = []
        for c in range(_HCH):
            o = hb + c * _L
            lv16 = lv[pl.ds(o, _L)]
            vv16 = vv[o // 128, pl.ds(o % 128, _L)]
            for l in range(_L):
                m = c * _L + l
                copies.append(pltpu.async_copy(
                    x_hbm.at[pl.ds(lv16[l], 1)], xi.at[pl.ds(m, 1)], sem))
                copies.append(pltpu.async_copy(
                    a_hbm.at[pl.ds(vv16[l], 1)], aj.at[pl.ds(m, 1)], sem))
        if h == 0:
            for cp in bcopies:
                cp.wait()
        for cp in copies:
            cp.wait()

        for c in range(_HCH):
            rows = c * _L + lax.iota(jnp.int32, _L)
            bv = bj[pl.ds(hb + c * _L, _L)]
            d0 = plsc.load_gather(xi, [rows, k0]) - bv
            d1 = plsc.load_gather(xi, [rows, k1]) - bv
            d2 = plsc.load_gather(xi, [rows, k2]) - bv
            dist2 = d0 * d0 + d1 * d1 + d2 * d2
            g0 = plsc.load_gather(aj, [rows, k0])
            g1 = plsc.load_gather(aj, [rows, k1])
            g2 = plsc.load_gather(aj, [rows, k2])
            sal2 = g0 * g0 + g1 * g1 + g2 * g2
            t = _sqrt16(jnp.minimum(dist2 * sal2, 3.0e38))
            ov[pl.ds(hb + c * _L, _L)] = 1.0 / (1.0 + jnp.exp(-t))

    for j in range(_JROWS):
        pltpu.sync_copy(ov.at[pl.ds(j * 128, 128)],
                        out_hbm.at[pl.ds(base + j * 128, 128)])


_ipm = functools.partial(
    pl.kernel,
    mesh=plsc.VectorSubcoreMesh(core_axis_name="c", subcore_axis_name="s"),
    out_type=jax.ShapeDtypeStruct((_B,), jnp.float32),
    compiler_params=pltpu.CompilerParams(needs_layout_passes=False),
    scratch_types=[
        pltpu.VMEM((_PER_W,), jnp.int32),
        pltpu.VMEM((_JROWS, 128), jnp.int32),
        pltpu.VMEM((_HALF, _DIM), jnp.float32),
        pltpu.VMEM((_HALF, _DIM), jnp.float32),
        pltpu.VMEM((_PER_W,), jnp.float32),
        pltpu.VMEM((_PER_W,), jnp.float32),
        pltpu.SemaphoreType.DMA,
    ],
)(_body)


def kernel(leg_ids, vote_ids, x, a, b):
    return _ipm(leg_ids, vote_ids, x, a, b)
